# baseline (speedup 1.0000x reference)
import functools

import jax
import jax.numpy as jnp
from jax import lax
from jax.experimental import pallas as pl

BETA = 0.25
_KT = 1024


_MT = 256


def _stage_body(r_ref, cb_ref, rn_ref, en_ref, idx_ref, zq_ref, *, m, k, d):
    r = r_ref[...]
    rn = rn_ref[...]
    nt = k // _KT

    def score_step(kt, carry):
        best_val, best_idx = carry
        cb_t = cb_ref[pl.ds(kt * _KT, _KT), :]
        en_t = en_ref[:, pl.ds(kt * _KT, _KT)]
        mm = lax.dot_general(
            r, cb_t, (((1,), (1,)), ((), ())),
            preferred_element_type=jnp.float32)
        s = (rn + en_t) - 2.0 * mm
        m_t = jnp.min(s, axis=1, keepdims=True)
        ii = lax.broadcasted_iota(jnp.int32, s.shape, 1) + kt * _KT
        cand = jnp.where(s == m_t, ii, jnp.int32(2**30))
        i_t = jnp.min(cand, axis=1, keepdims=True)
        upd = m_t < best_val
        return (jnp.where(upd, m_t, best_val), jnp.where(upd, i_t, best_idx))

    best_val = jnp.full((m, 1), jnp.inf, dtype=jnp.float32)
    best_idx = jnp.zeros((m, 1), dtype=jnp.int32)
    best_val, best_idx = lax.fori_loop(
        0, nt, score_step, (best_val, best_idx), unroll=False)

    idx_ref[...] = best_idx

    def gather_step(kt, zq):
        cb_t = cb_ref[pl.ds(kt * _KT, _KT), :]
        ii = lax.broadcasted_iota(jnp.int32, (m, _KT), 1) + kt * _KT
        oh = (best_idx == ii).astype(jnp.float32)
        return zq + lax.dot_general(
            oh, cb_t, (((1,), (0,)), ((), ())),
            preferred_element_type=jnp.float32,
            precision=lax.Precision.HIGHEST)

    zq_ref[...] = lax.fori_loop(
        0, nt, gather_step, jnp.zeros((m, d), dtype=jnp.float32),
        unroll=False)


def _stage(r, cb, rn, en):
    n, d = r.shape
    k = cb.shape[0]
    body = functools.partial(_stage_body, m=_MT, k=k, d=d)
    return pl.pallas_call(
        body,
        grid=(n // _MT,),
        in_specs=[
            pl.BlockSpec((_MT, d), lambda i: (i, 0)),
            pl.BlockSpec((k, d), lambda i: (0, 0)),
            pl.BlockSpec((_MT, 1), lambda i: (i, 0)),
            pl.BlockSpec((1, k), lambda i: (0, 0)),
        ],
        out_specs=(
            pl.BlockSpec((_MT, 1), lambda i: (i, 0)),
            pl.BlockSpec((_MT, d), lambda i: (i, 0)),
        ),
        out_shape=(
            jax.ShapeDtypeStruct((n, 1), jnp.int32),
            jax.ShapeDtypeStruct((n, d), jnp.float32),
        ),
    )(r, cb, rn, en)


def kernel(z, codebooks):
    B, T, D = z.shape
    zf = z.reshape(-1, D)
    residual = zf
    z_q = jnp.zeros_like(zf)
    losses = []
    inds = []
    for q in range(codebooks.shape[0]):
        emb = codebooks[q]
        rn = jnp.sum(residual ** 2, axis=1, keepdims=True)
        en = jnp.sum(emb ** 2, axis=1)
        idx2, zq = _stage(residual, emb, rn, en.reshape(1, -1))
        idx = idx2.reshape(-1)
        m = jnp.mean((zq - residual) ** 2)
        loss = m + BETA * m
        zq_st = residual + (zq - residual)
        residual_new = residual - zq_st
        z_q = z_q + zq_st
        losses.append(loss)
        inds.append(idx)
        residual = residual_new
    mean_losses = jnp.stack(losses).mean()
    all_min_encoding_indices = jnp.stack(inds, axis=1)
    return z_q.reshape(B, T, D), mean_losses, all_min_encoding_indices


# speedup vs baseline: 1.9427x; 1.9427x over previous
import functools

import jax
import jax.numpy as jnp
from jax import lax
from jax.experimental import pallas as pl
from jax.experimental.pallas import tpu as pltpu
from jax.experimental.pallas import tpu_sc as plsc

BETA = 0.25
_KT = 1024

_NW = 32


def _make_sc_gather(v, d, b):
    b_per_w = b // _NW
    mesh = plsc.VectorSubcoreMesh(core_axis_name="c", subcore_axis_name="s")

    @functools.partial(
        pl.kernel, mesh=mesh,
        out_type=jax.ShapeDtypeStruct((b, d), jnp.float32),
        scratch_types=[
            pltpu.VMEM((b_per_w,), jnp.int32),
            pltpu.VMEM((b_per_w, d), jnp.float32),
            pltpu.SemaphoreType.DMA,
        ],
    )
    def gather_kernel(table_hbm, idx_hbm, out_hbm, idx_v, rows_v, sem):
        wid = lax.axis_index("s") * 2 + lax.axis_index("c")
        base = wid * b_per_w
        pltpu.sync_copy(idx_hbm.at[pl.ds(base, b_per_w)], idx_v)
        pltpu.async_copy(table_hbm.at[idx_v], rows_v, sem).wait()
        pltpu.sync_copy(rows_v, out_hbm.at[pl.ds(base, b_per_w)])

    return gather_kernel


_MT = 256


def _stage_body(r_ref, cb_ref, en_ref, idx_ref, *, m, k, d):
    r = r_ref[...]
    rn = jnp.sum(r ** 2, axis=1, keepdims=True)
    nt = k // _KT

    def score_step(kt, carry):
        best_val, best_idx = carry
        cb_t = cb_ref[pl.ds(kt * _KT, _KT), :]
        en_t = en_ref[:, pl.ds(kt * _KT, _KT)]
        mm = lax.dot_general(
            r, cb_t, (((1,), (1,)), ((), ())),
            preferred_element_type=jnp.float32)
        s = (rn + en_t) - 2.0 * mm
        m_t = jnp.min(s, axis=1, keepdims=True)
        ii = lax.broadcasted_iota(jnp.int32, s.shape, 1) + kt * _KT
        cand = jnp.where(s == m_t, ii, jnp.int32(2**30))
        i_t = jnp.min(cand, axis=1, keepdims=True)
        upd = m_t < best_val
        return (jnp.where(upd, m_t, best_val), jnp.where(upd, i_t, best_idx))

    best_val = jnp.full((m, 1), jnp.inf, dtype=jnp.float32)
    best_idx = jnp.zeros((m, 1), dtype=jnp.int32)
    best_val, best_idx = lax.fori_loop(
        0, nt, score_step, (best_val, best_idx), unroll=False)

    idx_ref[...] = best_idx


def _stage(r, cb, en):
    n, d = r.shape
    k = cb.shape[0]
    body = functools.partial(_stage_body, m=_MT, k=k, d=d)
    return pl.pallas_call(
        body,
        grid=(n // _MT,),
        in_specs=[
            pl.BlockSpec((_MT, d), lambda i: (i, 0)),
            pl.BlockSpec((k, d), lambda i: (0, 0)),
            pl.BlockSpec((1, k), lambda i: (0, 0)),
        ],
        out_specs=pl.BlockSpec((_MT, 1), lambda i: (i, 0)),
        out_shape=jax.ShapeDtypeStruct((n, 1), jnp.int32),
    )(r, cb, en)


def kernel(z, codebooks):
    B, T, D = z.shape
    zf = z.reshape(-1, D)
    residual = zf
    z_q = jnp.zeros_like(zf)
    losses = []
    inds = []
    n = zf.shape[0]
    sc_gather = _make_sc_gather(codebooks.shape[1], D, n)
    for q in range(codebooks.shape[0]):
        emb = codebooks[q]
        en = jnp.sum(emb ** 2, axis=1)
        idx2 = _stage(residual, emb, en.reshape(1, -1))
        idx = idx2.reshape(-1)
        zq = sc_gather(emb, idx)
        m = jnp.mean((zq - residual) ** 2)
        loss = m + BETA * m
        zq_st = residual + (zq - residual)
        residual_new = residual - zq_st
        z_q = z_q + zq_st
        losses.append(loss)
        inds.append(idx)
        residual = residual_new
    mean_losses = jnp.stack(losses).mean()
    all_min_encoding_indices = jnp.stack(inds, axis=1)
    return z_q.reshape(B, T, D), mean_losses, all_min_encoding_indices


# speedup vs baseline: 3.0354x; 1.5624x over previous
import functools

import jax
import jax.numpy as jnp
from jax import lax
from jax.experimental import pallas as pl
from jax.experimental.pallas import tpu as pltpu
from jax.experimental.pallas import tpu_sc as plsc

BETA = 0.25
_KT = 1024

_NW = 32


def _make_sc_gather(v, d, b):
    b_per_w = b // _NW
    mesh = plsc.VectorSubcoreMesh(core_axis_name="c", subcore_axis_name="s")

    @functools.partial(
        pl.kernel, mesh=mesh,
        out_type=jax.ShapeDtypeStruct((b, d), jnp.float32),
        scratch_types=[
            pltpu.VMEM((b_per_w,), jnp.int32),
            pltpu.VMEM((b_per_w, d), jnp.float32),
            pltpu.SemaphoreType.DMA,
        ],
    )
    def gather_kernel(table_hbm, idx_hbm, out_hbm, idx_v, rows_v, sem):
        wid = lax.axis_index("s") * 2 + lax.axis_index("c")
        base = wid * b_per_w
        pltpu.sync_copy(idx_hbm.at[pl.ds(base, b_per_w)], idx_v)
        pltpu.async_copy(table_hbm.at[idx_v], rows_v, sem).wait()
        pltpu.sync_copy(rows_v, out_hbm.at[pl.ds(base, b_per_w)])

    return gather_kernel


_MT = 256


def _stage_body(r_ref, cb_ref, en_ref, idx_ref, *, m, k, d):
    r = r_ref[...]
    rn = jnp.sum(r ** 2, axis=1, keepdims=True)
    r2 = r + r
    fiota = lax.broadcasted_iota(jnp.int32, (1, _KT), 1).astype(jnp.float32)
    nt = k // _KT

    mins = []
    args = []
    for kt in range(nt):
        cb_t = cb_ref[pl.ds(kt * _KT, _KT), :]
        en_t = en_ref[:, pl.ds(kt * _KT, _KT)]
        mm2 = lax.dot_general(
            r2, cb_t, (((1,), (1,)), ((), ())),
            preferred_element_type=jnp.float32)
        s = (rn + en_t) - mm2
        m_t = jnp.min(s, axis=1, keepdims=True)
        cand = jnp.where(s == m_t, fiota, jnp.float32(8192.0))
        i_t = jnp.min(cand, axis=1, keepdims=True) + jnp.float32(kt * _KT)
        mins.append(m_t)
        args.append(i_t)

    best_val, best_idx = mins[0], args[0]
    for kt in range(1, nt):
        upd = mins[kt] < best_val
        best_val = jnp.where(upd, mins[kt], best_val)
        best_idx = jnp.where(upd, args[kt], best_idx)

    idx_ref[...] = best_idx.astype(jnp.int32)


def _stage(r, cb, en):
    n, d = r.shape
    k = cb.shape[0]
    body = functools.partial(_stage_body, m=_MT, k=k, d=d)
    return pl.pallas_call(
        body,
        grid=(n // _MT,),
        in_specs=[
            pl.BlockSpec((_MT, d), lambda i: (i, 0)),
            pl.BlockSpec((k, d), lambda i: (0, 0)),
            pl.BlockSpec((1, k), lambda i: (0, 0)),
        ],
        out_specs=pl.BlockSpec((_MT, 1), lambda i: (i, 0)),
        out_shape=jax.ShapeDtypeStruct((n, 1), jnp.int32),
    )(r, cb, en)


def kernel(z, codebooks):
    B, T, D = z.shape
    zf = z.reshape(-1, D)
    residual = zf
    z_q = jnp.zeros_like(zf)
    losses = []
    inds = []
    n = zf.shape[0]
    sc_gather = _make_sc_gather(codebooks.shape[1], D, n)
    for q in range(codebooks.shape[0]):
        emb = codebooks[q]
        en = jnp.sum(emb ** 2, axis=1)
        idx2 = _stage(residual, emb, en.reshape(1, -1))
        idx = idx2.reshape(-1)
        zq = sc_gather(emb, idx)
        m = jnp.mean((zq - residual) ** 2)
        loss = m + BETA * m
        zq_st = residual + (zq - residual)
        residual_new = residual - zq_st
        z_q = z_q + zq_st
        losses.append(loss)
        inds.append(idx)
        residual = residual_new
    mean_losses = jnp.stack(losses).mean()
    all_min_encoding_indices = jnp.stack(inds, axis=1)
    return z_q.reshape(B, T, D), mean_losses, all_min_encoding_indices
